# baseline (speedup 1.0000x reference)
import functools

import jax
import jax.numpy as jnp
from jax import lax
from jax.experimental import pallas as pl
from jax.experimental.pallas import tpu as pltpu
from jax.experimental.pallas import tpu_sc as plsc

_NC = 2
_NS = 16
_L = 16
_NW = _NC * _NS


def kernel(opinions, weights):
    E, B, d = opinions.shape
    b_per_w = B // _NW
    CH = 64
    n_ch = b_per_w // CH
    n_grp = b_per_w // _L
    d_vecs = d // _L

    mesh = plsc.VectorSubcoreMesh(core_axis_name="c", subcore_axis_name="s")

    @functools.partial(
        pl.kernel,
        out_type=jax.ShapeDtypeStruct((B * d,), jnp.float32),
        mesh=mesh,
        scratch_types=[
            pltpu.VMEM((b_per_w * E,), jnp.float32),
            pltpu.VMEM((b_per_w,), jnp.int32),
            pltpu.VMEM((E * d,), jnp.float32),
            pltpu.VMEM((CH * d,), jnp.float32),
            pltpu.VMEM((CH * d,), jnp.float32),
            pltpu.SemaphoreType.DMA,
        ],
        compiler_params=pltpu.CompilerParams(needs_layout_passes=False),
    )
    def k(op_hbm, w_hbm, out_hbm, w_v, idx_v, table_v, rows_a, rows_b, wsem):
        wid = lax.axis_index("s") * _NC + lax.axis_index("c")
        base = wid * b_per_w

        pltpu.sync_copy(w_hbm.at[pl.ds(base * E, b_per_w * E)], w_v)
        pltpu.sync_copy(op_hbm.at[pl.ds(0, E * d)], table_v)

        iota = lax.iota(jnp.int32, _L)

        def argmax_group(g, _):
            fvec = (g * _L + iota) * E
            best_v = plsc.load_gather(w_v, [fvec])
            best_i = jnp.zeros((_L,), jnp.int32)
            for e in range(1, E):
                v = plsc.load_gather(w_v, [fvec + e])
                p = v > best_v
                best_v = jnp.where(p, v, best_v)
                best_i = jnp.where(p, e * d, best_i)
            idx_v[pl.ds(g * _L, _L)] = best_i
            return 0

        lax.fori_loop(0, n_grp, argmax_group, 0)

        g_per_ch = CH // _L
        bufs = [rows_a, rows_b]
        writes = [None, None]
        for c in range(n_ch):
            b = c & 1
            if writes[b] is not None:
                writes[b].wait()
            buf = bufs[b]

            @plsc.parallel_loop(0, g_per_ch)
            def _chunk(gg):
                ev = idx_v[pl.ds((c * g_per_ch + gg) * _L, _L)]
                for u in range(_L):
                    src = jnp.take_along_axis(
                        ev, jnp.full((_L,), u, jnp.int32), axis=0,
                        mode=lax.GatherScatterMode.PROMISE_IN_BOUNDS) + iota
                    dst = (gg * _L + u) * d

                    @plsc.parallel_loop(0, d_vecs, unroll=8)
                    def _cols(j):
                        buf[pl.ds(dst + j * _L, _L)] = plsc.load_gather(
                            table_v, [src + j * _L])

            writes[b] = pltpu.async_copy(
                buf, out_hbm.at[pl.ds((base + c * CH) * d, CH * d)], wsem)
        for w in writes:
            if w is not None:
                w.wait()

    out = k(opinions.reshape(E * B * d), weights.reshape(B * E))
    return out.reshape(B, d)


# speedup vs baseline: 1.0044x; 1.0044x over previous
import functools

import jax
import jax.numpy as jnp
from jax import lax
from jax.experimental import pallas as pl
from jax.experimental.pallas import tpu as pltpu
from jax.experimental.pallas import tpu_sc as plsc

_NC = 2
_NS = 16
_L = 16
_NW = _NC * _NS


def kernel(opinions, weights):
    E, B, d = opinions.shape
    b_per_w = B // _NW
    CH = 64
    n_ch = b_per_w // CH
    n_grp = b_per_w // _L
    d_vecs = d // _L

    mesh = plsc.VectorSubcoreMesh(core_axis_name="c", subcore_axis_name="s")

    @functools.partial(
        pl.kernel,
        out_type=jax.ShapeDtypeStruct((B * d,), jnp.float32),
        mesh=mesh,
        scratch_types=[
            pltpu.VMEM((b_per_w * E,), jnp.float32),
            pltpu.VMEM((b_per_w,), jnp.int32),
            pltpu.VMEM((E * d,), jnp.float32),
            pltpu.VMEM((CH * d,), jnp.float32),
            pltpu.VMEM((CH * d,), jnp.float32),
            pltpu.SemaphoreType.DMA,
        ],
        compiler_params=pltpu.CompilerParams(needs_layout_passes=False),
    )
    def k(op_hbm, w_hbm, out_hbm, w_v, idx_v, table_v, rows_a, rows_b, wsem):
        wid = lax.axis_index("s") * _NC + lax.axis_index("c")
        base = wid * b_per_w

        pltpu.sync_copy(w_hbm.at[pl.ds(base * E, b_per_w * E)], w_v)
        pltpu.sync_copy(op_hbm.at[pl.ds(0, E * d)], table_v)

        iota = lax.iota(jnp.int32, _L)

        def argmax_group(g, _):
            fvec = (g * _L + iota) * E
            best_v = plsc.load_gather(w_v, [fvec])
            best_i = jnp.zeros((_L,), jnp.int32)
            for e in range(1, E):
                v = plsc.load_gather(w_v, [fvec + e])
                p = v > best_v
                best_v = jnp.where(p, v, best_v)
                best_i = jnp.where(p, e * d, best_i)
            idx_v[pl.ds(g * _L, _L)] = best_i
            return 0

        lax.fori_loop(0, n_grp, argmax_group, 0)

        g_per_ch = CH // _L
        bufs = [rows_a, rows_b]
        writes = [None, None]
        for c in range(n_ch):
            b = c & 1
            if writes[b] is not None:
                writes[b].wait()
            buf = bufs[b]

            @plsc.parallel_loop(0, g_per_ch)
            def _chunk(gg):
                ev = idx_v[pl.ds((c * g_per_ch + gg) * _L, _L)]
                for u in range(_L):
                    src = pl.multiple_of(ev[u], 256)
                    dst = (gg * _L + u) * d

                    @plsc.parallel_loop(0, d_vecs, unroll=8)
                    def _cols(j):
                        buf[pl.ds(dst + j * _L, _L)] = (
                            table_v[pl.ds(src + j * _L, _L)])

            writes[b] = pltpu.async_copy(
                buf, out_hbm.at[pl.ds((base + c * CH) * d, CH * d)], wsem)
        for w in writes:
            if w is not None:
                w.wait()

    out = k(opinions.reshape(E * B * d), weights.reshape(B * E))
    return out.reshape(B, d)


# speedup vs baseline: 2.7188x; 2.7070x over previous
import functools

import jax
import jax.numpy as jnp
from jax import lax
from jax.experimental import pallas as pl
from jax.experimental.pallas import tpu as pltpu
from jax.experimental.pallas import tpu_sc as plsc

_NC = 2
_NS = 16
_L = 16
_NW = _NC * _NS


def kernel(opinions, weights):
    E, B, d = opinions.shape
    b_per_w = B // _NW
    CH = 32
    NBUF = 4
    DEPTH = 3
    n_ch = b_per_w // CH
    n_grp = b_per_w // _L

    mesh = plsc.VectorSubcoreMesh(core_axis_name="c", subcore_axis_name="s")

    @functools.partial(
        pl.kernel,
        out_type=jax.ShapeDtypeStruct((B, d), jnp.float32),
        mesh=mesh,
        scratch_types=[
            pltpu.VMEM((b_per_w * E,), jnp.float32),
            pltpu.VMEM((b_per_w,), jnp.int32),
        ] + [pltpu.VMEM((CH, d), jnp.float32) for _ in range(NBUF)]
          + [pltpu.SemaphoreType.DMA for _ in range(2 * NBUF)],
        compiler_params=pltpu.CompilerParams(needs_layout_passes=False),
    )
    def k(op_hbm, w_hbm, out_hbm, w_v, idx_v, *bufs_and_sems):
        bufs = bufs_and_sems[:NBUF]
        gsems = bufs_and_sems[NBUF:2 * NBUF]
        wsems = bufs_and_sems[2 * NBUF:]
        wid = lax.axis_index("s") * _NC + lax.axis_index("c")
        base = wid * b_per_w

        pltpu.sync_copy(w_hbm.at[pl.ds(base * E, b_per_w * E)], w_v)

        iota = lax.iota(jnp.int32, _L)

        def argmax_group(g, _):
            fvec = (g * _L + iota) * E
            best_v = plsc.load_gather(w_v, [fvec])
            best_i = jnp.zeros((_L,), jnp.int32)
            for e in range(1, E):
                v = plsc.load_gather(w_v, [fvec + e])
                p = v > best_v
                best_v = jnp.where(p, v, best_v)
                best_i = jnp.where(p, e, best_i)
            idx_v[pl.ds(g * _L, _L)] = best_i
            return 0

        lax.fori_loop(0, n_grp, argmax_group, 0)

        def start_gather(c, b):
            return pltpu.async_copy(
                op_hbm.at[idx_v.at[pl.ds(c * CH, CH)]], bufs[b], gsems[b])

        gets = [None] * NBUF
        writes = [None] * NBUF
        for c in range(min(DEPTH, n_ch)):
            gets[c % NBUF] = start_gather(c, c % NBUF)
        for c in range(n_ch):
            b = c % NBUF
            gets[b].wait()
            writes[b] = pltpu.async_copy(
                bufs[b], out_hbm.at[pl.ds(base + c * CH, CH)], wsems[b])
            nc = c + DEPTH
            if nc < n_ch:
                nb = nc % NBUF
                if writes[nb] is not None:
                    writes[nb].wait()
                    writes[nb] = None
                gets[nb] = start_gather(nc, nb)
        for w in writes:
            if w is not None:
                w.wait()

    return k(opinions.reshape(E * B, d), weights.reshape(B * E))


# speedup vs baseline: 7.4118x; 2.7261x over previous
import functools

import jax
import jax.numpy as jnp
from jax import lax
from jax.experimental import pallas as pl
from jax.experimental.pallas import tpu as pltpu
from jax.experimental.pallas import tpu_sc as plsc

_NC = 2
_NS = 16
_L = 16
_NW = _NC * _NS


def _sc_argmax(weights_flat, B, E):
    b_per_w = B // _NW
    n_grp = b_per_w // _L
    mesh = plsc.VectorSubcoreMesh(core_axis_name="c", subcore_axis_name="s")

    @functools.partial(
        pl.kernel,
        out_type=jax.ShapeDtypeStruct((B,), jnp.int32),
        mesh=mesh,
        scratch_types=[
            pltpu.VMEM((b_per_w * E,), jnp.float32),
            pltpu.VMEM((b_per_w,), jnp.int32),
        ],
        compiler_params=pltpu.CompilerParams(needs_layout_passes=False),
    )
    def k(w_hbm, out_hbm, w_v, idx_v):
        wid = lax.axis_index("s") * _NC + lax.axis_index("c")
        base = wid * b_per_w

        pltpu.sync_copy(w_hbm.at[pl.ds(base * E, b_per_w * E)], w_v)

        iota = lax.iota(jnp.int32, _L)

        def argmax_group(g, _):
            fvec = (g * _L + iota) * E
            best_v = plsc.load_gather(w_v, [fvec])
            best_i = jnp.zeros((_L,), jnp.int32)
            for e in range(1, E):
                v = plsc.load_gather(w_v, [fvec + e])
                p = v > best_v
                best_v = jnp.where(p, v, best_v)
                best_i = jnp.where(p, e, best_i)
            idx_v[pl.ds(g * _L, _L)] = best_i
            return 0

        lax.fori_loop(0, n_grp, argmax_group, 0)
        pltpu.sync_copy(idx_v, out_hbm.at[pl.ds(base, b_per_w)])

    return k(weights_flat)


def _tc_expand(elems, op_cat, B, E, d):
    BLK = 1024
    NB = B // BLK

    def body(e_ref, t_ref, o_ref):
        e = e_ref[0, 0, :].reshape(BLK, 1)
        acc = jnp.broadcast_to(t_ref[0:1, :], (BLK, d))
        for k in range(1, E):
            acc = jnp.where(
                e == k, jnp.broadcast_to(t_ref[k:k + 1, :], (BLK, d)), acc)
        o_ref[...] = acc

    return pl.pallas_call(
        body,
        grid=(NB,),
        in_specs=[
            pl.BlockSpec((1, 1, BLK), lambda i: (i, 0, 0)),
            pl.BlockSpec((E, d), lambda i: (0, 0)),
        ],
        out_specs=pl.BlockSpec((BLK, d), lambda i: (i, 0)),
        out_shape=jax.ShapeDtypeStruct((B, d), jnp.float32),
    )(elems.reshape(NB, 1, BLK), op_cat)


def kernel(opinions, weights):
    E, B, d = opinions.shape
    op_cat = opinions.reshape(E * B, d)
    elems = _sc_argmax(weights.reshape(B * E), B, E)
    return _tc_expand(elems, op_cat, B, E, d)
